# baseline (speedup 1.0000x reference)
import functools

import jax
import jax.numpy as jnp
from jax import lax
from jax.experimental import pallas as pl
from jax.experimental.pallas import tpu as pltpu
from jax.experimental.pallas import tpu_sc as plsc

_NN = 1024
_NE = 2048
_HID = 512
_HEADS = 8
_DH = 64
_LAYERS = 3
_NG = 16
_NC = 2
_NS = 16

_f32 = jnp.float32


def _dotT(a, b):
    return lax.dot_general(a, b, (((1,), (1,)), ((), ())),
                           preferred_element_type=_f32)


def _dot(a, b):
    return lax.dot_general(a, b, (((1,), (0,)), ((), ())),
                           preferred_element_type=_f32)


def _gelu(x):
    return x * 0.5 * (1.0 + lax.erf(x * (2.0 ** -0.5)))


def _ln(x, g, b):
    m = jnp.mean(x, axis=-1, keepdims=True)
    xc = x - m
    v = jnp.mean(xc * xc, axis=-1, keepdims=True)
    return xc * lax.rsqrt(v + 1e-5) * g + b


def _embed_body(ea_ref, bew_ref, beb_ref, whw_ref, whb_ref,
                dstc_ref, dstr_ref, srcc_ref, srcr_ref,
                h_ref, cnt_ref):
    t = _gelu(_dotT(ea_ref[...], bew_ref[...]) + beb_ref[...])
    h_ref[...] = _dotT(t, whw_ref[...]) + whb_ref[...]
    dc = dstc_ref[...]
    dr = dstr_ref[...]
    sc = srcc_ref[...]
    sr = srcr_ref[...]
    blocks = []
    nb = 8
    rb = _NE // nb
    for b in range(nb):
        dcb = dc[b * rb:(b + 1) * rb]
        scb = sc[b * rb:(b + 1) * rb]
        m = ((dcb == dr) & (scb != sr)).astype(_f32)
        blocks.append(jnp.sum(m, axis=-1, keepdims=True))
    cnt_ref[...] = jnp.concatenate(blocks, axis=0)


def _embed(edge_attr, be_w, be_b, wh_w, wh_b, dst, src):
    return pl.pallas_call(
        _embed_body,
        out_shape=(jax.ShapeDtypeStruct((_NE, _HID), _f32),
                   jax.ShapeDtypeStruct((_NE, 1), _f32)),
    )(edge_attr, be_w, be_b.reshape(1, -1), wh_w, wh_b.reshape(1, -1),
      dst.reshape(_NE, 1), dst.reshape(1, _NE),
      src.reshape(_NE, 1), src.reshape(1, _NE))


_LANES = 16
_NW = _NC * _NS
_CH = 64


def _bcast_lane(vec, k):
    idx = jnp.full((_LANES, 1), k, jnp.int32)
    return lax.gather(vec, idx,
                      lax.GatherDimensionNumbers(offset_dims=(),
                                                 collapsed_slice_dims=(0,),
                                                 start_index_map=(0,)),
                      (1,), mode=lax.GatherScatterMode.PROMISE_IN_BOUNDS)


_SC_PARAMS = pltpu.CompilerParams(needs_layout_passes=False)
_G = _NE // _LANES


def _sc_agg_body(ht_hbm, dst_hbm, cid_hbm, zs_hbm, zp_hbm,
                 sgt_hbm, pgt_hbm, hbuf, opbuf, dst_v, cid_v, tab_s, tab_p):
    c = lax.axis_index("c")
    s = lax.axis_index("s")
    col0 = (s * _NC + c) * _LANES
    pltpu.sync_copy(dst_hbm, dst_v)
    pltpu.sync_copy(cid_hbm, cid_v)
    pltpu.sync_copy(ht_hbm.at[pl.ds(col0, _LANES)], hbuf)
    pltpu.sync_copy(zs_hbm, tab_s)
    pltpu.sync_copy(zp_hbm, tab_p)
    lanes = lax.iota(jnp.int32, _LANES)

    def scat(g, cy):
        dvec = dst_v[pl.ds(g * _LANES, _LANES)]
        cvec = cid_v[pl.ds(g * _LANES, _LANES)]
        for k in range(_LANES):
            dcol = _bcast_lane(dvec, k)
            ccol = _bcast_lane(cvec, k)
            ev = jnp.full((_LANES,), g * _LANES + k, jnp.int32)
            hv = plsc.load_gather(hbuf, [lanes, ev])
            plsc.addupdate_scatter(tab_s, [lanes, dcol], hv)
            plsc.addupdate_scatter(tab_p, [lanes, ccol], hv)
        return cy

    lax.fori_loop(0, _G, scat, 0)

    def gath(g, cy):
        dvec = dst_v[pl.ds(g * _LANES, _LANES)]
        cvec = cid_v[pl.ds(g * _LANES, _LANES)]
        for k in range(_LANES):
            dcol = _bcast_lane(dvec, k)
            ccol = _bcast_lane(cvec, k)
            ev = jnp.full((_LANES,), g * _LANES + k, jnp.int32)
            sv = plsc.load_gather(tab_s, [lanes, dcol])
            pv = plsc.load_gather(tab_p, [lanes, ccol])
            plsc.store_scatter(hbuf, [lanes, ev], sv)
            plsc.store_scatter(opbuf, [lanes, ev], pv)
        return cy

    lax.fori_loop(0, _G, gath, 0)
    pltpu.sync_copy(hbuf, sgt_hbm.at[pl.ds(col0, _LANES)])
    pltpu.sync_copy(opbuf, pgt_hbm.at[pl.ds(col0, _LANES)])


def _sc_aggregate(ht, dst, cid, zs, zp):
    return pl.kernel(
        _sc_agg_body,
        out_type=(jax.ShapeDtypeStruct((_HID, _NE), _f32),
                  jax.ShapeDtypeStruct((_HID, _NE), _f32)),
        mesh=plsc.VectorSubcoreMesh(core_axis_name="c", subcore_axis_name="s"),
        scratch_types=[
            pltpu.VMEM((_LANES, _NE), _f32),
            pltpu.VMEM((_LANES, _NE), _f32),
            pltpu.VMEM((_NE,), jnp.int32),
            pltpu.VMEM((_NE,), jnp.int32),
            pltpu.VMEM((_LANES, _NN), _f32),
            pltpu.VMEM((_LANES, _NE), _f32),
        ],
        compiler_params=_SC_PARAMS,
    )(ht, dst, cid, zs, zp)


def _sc_nodes_body(ht_hbm, dst_hbm, zs_hbm, hibt_hbm, hbuf, dst_v, tab_s):
    c = lax.axis_index("c")
    s = lax.axis_index("s")
    col0 = (s * _NC + c) * _LANES
    pltpu.sync_copy(dst_hbm, dst_v)
    pltpu.sync_copy(ht_hbm.at[pl.ds(col0, _LANES)], hbuf)
    pltpu.sync_copy(zs_hbm, tab_s)
    lanes = lax.iota(jnp.int32, _LANES)

    def scat(g, cy):
        dvec = dst_v[pl.ds(g * _LANES, _LANES)]
        for k in range(_LANES):
            dcol = _bcast_lane(dvec, k)
            ev = jnp.full((_LANES,), g * _LANES + k, jnp.int32)
            hv = plsc.load_gather(hbuf, [lanes, ev])
            plsc.addupdate_scatter(tab_s, [lanes, dcol], hv)
        return cy

    lax.fori_loop(0, _G, scat, 0)
    pltpu.sync_copy(tab_s, hibt_hbm.at[pl.ds(col0, _LANES)])


def _sc_scatter_nodes(ht, dst, zs):
    return pl.kernel(
        _sc_nodes_body,
        out_type=jax.ShapeDtypeStruct((_HID, _NN), _f32),
        mesh=plsc.VectorSubcoreMesh(core_axis_name="c", subcore_axis_name="s"),
        scratch_types=[
            pltpu.VMEM((_LANES, _NE), _f32),
            pltpu.VMEM((_NE,), jnp.int32),
            pltpu.VMEM((_LANES, _NN), _f32),
        ],
        compiler_params=_SC_PARAMS,
    )(ht, dst, zs)


_QB = 512


def _mha_blocks(xn, w3, b3, woutr, n):
    ks = []
    vs = []
    for hh in range(_HEADS):
        ks.append(_dotT(xn, w3[1, hh]) + b3[1, hh][None, :])
        vs.append(_dotT(xn, w3[2, hh]) + b3[2, hh][None, :])
    scale = 1.0 / (_DH ** 0.5)
    blocks = []
    for qb in range(n // _QB):
        xb = xn[qb * _QB:(qb + 1) * _QB]
        acc = None
        for hh in range(_HEADS):
            qh = _dotT(xb, w3[0, hh]) + b3[0, hh][None, :]
            sc = _dotT(qh, ks[hh]) * scale
            m = jnp.max(sc, axis=-1, keepdims=True)
            p = jnp.exp(sc - m)
            a = p / jnp.sum(p, axis=-1, keepdims=True)
            oh = _dot(a, vs[hh])
            contrib = _dotT(oh, woutr[hh])
            acc = contrib if acc is None else acc + contrib
        blocks.append(acc)
    return jnp.concatenate(blocks, axis=0)


def _bond_layer_body(h_ref, sg_ref, pg_ref, cnt_ref, lng_ref, lnb_ref,
                     w3_ref, b3_ref, woutr_ref, outb_ref, upw_ref, upb_ref,
                     uplng_ref, uplnb_ref, ho_ref):
    h = h_ref[...]
    r = sg_ref[...] - pg_ref[...] - cnt_ref[...] * h
    xn = _ln(r, lng_ref[...], lnb_ref[...])
    attn = _mha_blocks(xn, w3_ref[...], b3_ref[...], woutr_ref[...], _NE)
    t = attn + outb_ref[...] + 2.0 * r
    u = _dotT(t, upw_ref[...]) + upb_ref[...]
    ho_ref[...] = _gelu(_ln(u, uplng_ref[...], uplnb_ref[...]))


def _bond_layer(h, sg, pg, cnt, lng, lnb, in_w, in_b, out_w, out_b,
                up_w, up_b, uplng, uplnb):
    w3 = in_w.reshape(3, _HEADS, _DH, _HID)
    b3 = in_b.reshape(3, _HEADS, _DH)
    woutr = out_w.reshape(_HID, _HEADS, _DH).transpose(1, 0, 2)
    return pl.pallas_call(
        _bond_layer_body,
        out_shape=jax.ShapeDtypeStruct((_NE, _HID), _f32),
    )(h, sg, pg, cnt, lng.reshape(1, -1), lnb.reshape(1, -1),
      w3, b3, woutr, out_b.reshape(1, -1), up_w, up_b.reshape(1, -1),
      uplng.reshape(1, -1), uplnb.reshape(1, -1))


def _atom_body(x_ref, hib_ref, aew_ref, aeb_ref, aelng_ref, aelnb_ref,
               fpwa_ref, fpwb_ref, fpb_ref, alng_ref, alnb_ref,
               a3_ref, ab3_ref, awoutr_ref, aoutb_ref, batch_ref,
               gpw1_ref, gpb1_ref, gplng_ref, gplnb_ref, gpw2_ref, gpb2_ref,
               out_ref):
    a = _gelu(_ln(_dotT(x_ref[...], aew_ref[...]) + aeb_ref[...],
                  aelng_ref[...], aelnb_ref[...]))
    xi = _dotT(a, fpwa_ref[...]) + _dotT(hib_ref[...], fpwb_ref[...]) \
        + fpb_ref[...]
    xn = _ln(xi, alng_ref[...], alnb_ref[...])
    attn = _mha_blocks(xn, a3_ref[...], ab3_ref[...], awoutr_ref[...], _NN)
    hi = attn + aoutb_ref[...] + 2.0 * xi
    gids = lax.broadcasted_iota(jnp.int32, (1, _NG), 1)
    g_oh = (batch_ref[...] == gids).astype(_f32)
    ssum = lax.dot_general(g_oh, hi, (((0,), (0,)), ((), ())),
                           preferred_element_type=_f32)
    ones = jnp.ones((_NN, 1), _f32)
    ccol = lax.dot_general(g_oh, ones, (((0,), (0,)), ((), ())),
                           preferred_element_type=_f32)
    hg = ssum / ccol
    hg2 = _gelu(_ln(_dotT(hg, gpw1_ref[...]) + gpb1_ref[...],
                    gplng_ref[...], gplnb_ref[...]))
    out_ref[...] = _dotT(hg2, gpw2_ref[...]) + gpb2_ref[...]


def _atom_graph(x, hib, batch, ae_w, ae_b, aelng, aelnb, fp_w, fp_b,
                alng, alnb, ain_w, ain_b, aout_w, aout_b,
                gp_w1, gp_b1, gplng, gplnb, gp_w2, gp_b2):
    a3 = ain_w.reshape(3, _HEADS, _DH, _HID)
    ab3 = ain_b.reshape(3, _HEADS, _DH)
    awoutr = aout_w.reshape(_HID, _HEADS, _DH).transpose(1, 0, 2)
    return pl.pallas_call(
        _atom_body,
        out_shape=jax.ShapeDtypeStruct((_NG, _HID), _f32),
    )(x, hib, ae_w, ae_b.reshape(1, -1), aelng.reshape(1, -1),
      aelnb.reshape(1, -1), fp_w[:, :_HID], fp_w[:, _HID:],
      fp_b.reshape(1, -1), alng.reshape(1, -1), alnb.reshape(1, -1),
      a3, ab3, awoutr, aout_b.reshape(1, -1), batch.reshape(_NN, 1),
      gp_w1, gp_b1.reshape(1, -1), gplng.reshape(1, -1),
      gplnb.reshape(1, -1), gp_w2, gp_b2.reshape(1, -1))


def _pair_ids(src, dst):
    key = src * _NN + dst
    order = jnp.argsort(key)
    ks = jnp.take(key, order)
    new = jnp.concatenate([jnp.ones((1,), jnp.int32),
                           (ks[1:] != ks[:-1]).astype(jnp.int32)])
    cids = jnp.cumsum(new) - 1
    return jnp.zeros((_NE,), jnp.int32).at[order].set(cids.astype(jnp.int32))


def kernel(x, edge_attr, edge_index, batch, be_w, be_b, wh_w, wh_b,
           bln_g, bln_b, bin_w, bin_b, bout_w, bout_b, bup_w, bup_b,
           bupln_g, bupln_b, ae_w, ae_b, aeln_g, aeln_b, fp_w, fp_b,
           aln_g, aln_b, ain_w, ain_b, aout_w, aout_b,
           gp_w1, gp_b1, gpln_g, gpln_b, gp_w2, gp_b2):
    src = edge_index[0]
    dst = edge_index[1]
    cid = _pair_ids(src, dst)
    zs = jnp.zeros((_LANES, _NN), _f32)
    zp = jnp.zeros((_LANES, _NE), _f32)

    h, cnt = _embed(edge_attr, be_w, be_b, wh_w, wh_b, dst, src)
    for t in range(_LAYERS):
        sgt, pgt = _sc_aggregate(h.T, dst, cid, zs, zp)
        h = _bond_layer(h, sgt.T, pgt.T, cnt, bln_g[t], bln_b[t], bin_w[t],
                        bin_b[t], bout_w[t], bout_b[t], bup_w[t], bup_b[t],
                        bupln_g[t], bupln_b[t])
    hibt = _sc_scatter_nodes(h.T, dst, zs)
    return _atom_graph(x, hibt.T, batch, ae_w, ae_b,
                       aeln_g, aeln_b, fp_w, fp_b,
                       aln_g, aln_b, ain_w, ain_b, aout_w, aout_b,
                       gp_w1, gp_b1, gpln_g, gpln_b, gp_w2, gp_b2)


# speedup vs baseline: 1.2842x; 1.2842x over previous
import functools

import jax
import jax.numpy as jnp
from jax import lax
from jax.experimental import pallas as pl
from jax.experimental.pallas import tpu as pltpu
from jax.experimental.pallas import tpu_sc as plsc

_NN = 1024
_NE = 2048
_HID = 512
_HEADS = 8
_DH = 64
_LAYERS = 3
_NG = 16
_NC = 2
_NS = 16

_f32 = jnp.float32


def _dotT(a, b):
    return lax.dot_general(a, b, (((1,), (1,)), ((), ())),
                           preferred_element_type=_f32)


def _dot(a, b):
    return lax.dot_general(a, b, (((1,), (0,)), ((), ())),
                           preferred_element_type=_f32)


def _gelu(x):
    return x * 0.5 * (1.0 + lax.erf(x * (2.0 ** -0.5)))


def _ln(x, g, b):
    m = jnp.mean(x, axis=-1, keepdims=True)
    xc = x - m
    v = jnp.mean(xc * xc, axis=-1, keepdims=True)
    return xc * lax.rsqrt(v + 1e-5) * g + b


def _embed_body(ea_ref, bew_ref, beb_ref, whw_ref, whb_ref,
                dstc_ref, dstr_ref, srcc_ref, srcr_ref,
                h_ref, cnt_ref):
    t = _gelu(_dotT(ea_ref[...], bew_ref[...]) + beb_ref[...])
    h_ref[...] = _dotT(t, whw_ref[...]) + whb_ref[...]
    dc = dstc_ref[...]
    dr = dstr_ref[...]
    sc = srcc_ref[...]
    sr = srcr_ref[...]
    blocks = []
    nb = 8
    rb = _NE // nb
    for b in range(nb):
        dcb = dc[b * rb:(b + 1) * rb]
        scb = sc[b * rb:(b + 1) * rb]
        m = ((dcb == dr) & (scb != sr)).astype(_f32)
        blocks.append(jnp.sum(m, axis=-1, keepdims=True))
    cnt_ref[...] = jnp.concatenate(blocks, axis=0)


def _embed(edge_attr, be_w, be_b, wh_w, wh_b, dst, src):
    return pl.pallas_call(
        _embed_body,
        out_shape=(jax.ShapeDtypeStruct((_NE, _HID), _f32),
                   jax.ShapeDtypeStruct((_NE, 1), _f32)),
    )(edge_attr, be_w, be_b.reshape(1, -1), wh_w, wh_b.reshape(1, -1),
      dst.reshape(_NE, 1), dst.reshape(1, _NE),
      src.reshape(_NE, 1), src.reshape(1, _NE))


_LANES = 16
_NW = _NC * _NS
_CH = 64


def _bcast_lane(vec, k):
    idx = jnp.full((_LANES, 1), k, jnp.int32)
    return lax.gather(vec, idx,
                      lax.GatherDimensionNumbers(offset_dims=(),
                                                 collapsed_slice_dims=(0,),
                                                 start_index_map=(0,)),
                      (1,), mode=lax.GatherScatterMode.PROMISE_IN_BOUNDS)


_SC_PARAMS = pltpu.CompilerParams(needs_layout_passes=False)
_G = _NE // _LANES


_SC_PARAMS = pltpu.CompilerParams(needs_layout_passes=False)
_G = _NE // _LANES
_EPC = _NE // _NC
_GC = _EPC // _LANES
_EPW = _NE // (_NC * _NS)


def _bcast_lane(vec, k):
    idx = jnp.full((_LANES, 1), k, jnp.int32)
    return lax.gather(vec, idx,
                      lax.GatherDimensionNumbers(offset_dims=(),
                                                 collapsed_slice_dims=(0,),
                                                 start_index_map=(0,)),
                      (1,), mode=lax.GatherScatterMode.PROMISE_IN_BOUNDS)


def _sc_scat_body(ht_hbm, dst_hbm, cid_hbm, zs_hbm, zp_hbm,
                  s01_hbm, p01_hbm, hbuf, dst_v, cid_v, tab_s, tab_p):
    c = lax.axis_index("c")
    s = lax.axis_index("s")
    col0 = (s * _NC + c) * _LANES
    pltpu.sync_copy(dst_hbm, dst_v)
    pltpu.sync_copy(cid_hbm, cid_v)
    pltpu.sync_copy(ht_hbm.at[pl.ds(col0, _LANES)], hbuf)
    pltpu.sync_copy(zs_hbm, tab_s)
    pltpu.sync_copy(zp_hbm, tab_p)
    lanes = lax.iota(jnp.int32, _LANES)

    def scat(g, cy):
        dvec = dst_v[pl.ds(g * _LANES, _LANES)]
        cvec = cid_v[pl.ds(g * _LANES, _LANES)]
        for k in range(_LANES):
            dcol = _bcast_lane(dvec, k)
            ccol = _bcast_lane(cvec, k)
            ev = jnp.full((_LANES,), g * _LANES + k, jnp.int32)
            hv = plsc.load_gather(hbuf, [lanes, ev])
            plsc.addupdate_scatter(tab_s, [lanes, dcol], hv)
            plsc.addupdate_scatter(tab_p, [lanes, ccol], hv)
        return cy

    lax.fori_loop(c * _GC, (c + 1) * _GC, scat, 0)
    pltpu.sync_copy(tab_s, s01_hbm.at[pl.ds(c * _HID + col0, _LANES)])
    pltpu.sync_copy(tab_p, p01_hbm.at[pl.ds(c * _HID + col0, _LANES)])


def _sc_scatter_partials(ht, dst, cid, zs, zp):
    return pl.kernel(
        _sc_scat_body,
        out_type=(jax.ShapeDtypeStruct((_NC * _HID, _NN), _f32),
                  jax.ShapeDtypeStruct((_NC * _HID, _NE), _f32)),
        mesh=plsc.VectorSubcoreMesh(core_axis_name="c", subcore_axis_name="s"),
        scratch_types=[
            pltpu.VMEM((_LANES, _NE), _f32),
            pltpu.VMEM((_NE,), jnp.int32),
            pltpu.VMEM((_NE,), jnp.int32),
            pltpu.VMEM((_LANES, _NN), _f32),
            pltpu.VMEM((_LANES, _NE), _f32),
        ],
        compiler_params=_SC_PARAMS,
    )(ht, dst, cid, zs, zp)


def _sc_gath_body(dst_hbm, cid_hbm, s_hbm, p_hbm, sg_hbm, pg_hbm,
                  buf, idx1):
    c = lax.axis_index("c")
    s = lax.axis_index("s")
    base = (s * _NC + c) * _EPW
    pltpu.sync_copy(dst_hbm.at[pl.ds(base, _EPW)], idx1)
    pltpu.sync_copy(s_hbm.at[idx1], buf)
    pltpu.sync_copy(buf, sg_hbm.at[pl.ds(base, _EPW)])
    pltpu.sync_copy(cid_hbm.at[pl.ds(base, _EPW)], idx1)
    pltpu.sync_copy(p_hbm.at[idx1], buf)
    pltpu.sync_copy(buf, pg_hbm.at[pl.ds(base, _EPW)])


def _sc_gather_combine(dst, cid, s01, p01):
    return pl.kernel(
        _sc_gath_body,
        out_type=(jax.ShapeDtypeStruct((_NE, _HID), _f32),
                  jax.ShapeDtypeStruct((_NE, _HID), _f32)),
        mesh=plsc.VectorSubcoreMesh(core_axis_name="c", subcore_axis_name="s"),
        scratch_types=[
            pltpu.VMEM((_EPW, _HID), _f32),
            pltpu.VMEM((_EPW,), jnp.int32),
        ],
        compiler_params=_SC_PARAMS,
    )(dst, cid, s01, p01)


def _sc_aggregate(ht, dst, cid, zs, zp):
    s01t, p01t = _sc_scatter_partials(ht, dst, cid, zs, zp)
    s_full = (s01t[:_HID] + s01t[_HID:]).T
    p_full = (p01t[:_HID] + p01t[_HID:]).T
    return _sc_gather_combine(dst, cid, s_full, p_full)


def _sc_nodes_body(ht_hbm, dst_hbm, zs_hbm, hib01_hbm, hbuf, dst_v, tab_s):
    c = lax.axis_index("c")
    s = lax.axis_index("s")
    col0 = (s * _NC + c) * _LANES
    pltpu.sync_copy(dst_hbm, dst_v)
    pltpu.sync_copy(ht_hbm.at[pl.ds(col0, _LANES)], hbuf)
    pltpu.sync_copy(zs_hbm, tab_s)
    lanes = lax.iota(jnp.int32, _LANES)

    def scat(g, cy):
        dvec = dst_v[pl.ds(g * _LANES, _LANES)]
        for k in range(_LANES):
            dcol = _bcast_lane(dvec, k)
            ev = jnp.full((_LANES,), g * _LANES + k, jnp.int32)
            hv = plsc.load_gather(hbuf, [lanes, ev])
            plsc.addupdate_scatter(tab_s, [lanes, dcol], hv)
        return cy

    lax.fori_loop(c * _GC, (c + 1) * _GC, scat, 0)
    pltpu.sync_copy(tab_s, hib01_hbm.at[pl.ds(c * _HID + col0, _LANES)])


def _sc_scatter_nodes(ht, dst, zs):
    return pl.kernel(
        _sc_nodes_body,
        out_type=jax.ShapeDtypeStruct((_NC * _HID, _NN), _f32),
        mesh=plsc.VectorSubcoreMesh(core_axis_name="c", subcore_axis_name="s"),
        scratch_types=[
            pltpu.VMEM((_LANES, _NE), _f32),
            pltpu.VMEM((_NE,), jnp.int32),
            pltpu.VMEM((_LANES, _NN), _f32),
        ],
        compiler_params=_SC_PARAMS,
    )(ht, dst, zs)


_QB = 512


def _mha_blocks(xn, w3, b3, woutr, n):
    ks = []
    vs = []
    for hh in range(_HEADS):
        ks.append(_dotT(xn, w3[1, hh]) + b3[1, hh][None, :])
        vs.append(_dotT(xn, w3[2, hh]) + b3[2, hh][None, :])
    scale = 1.0 / (_DH ** 0.5)
    blocks = []
    for qb in range(n // _QB):
        xb = xn[qb * _QB:(qb + 1) * _QB]
        acc = None
        for hh in range(_HEADS):
            qh = _dotT(xb, w3[0, hh]) + b3[0, hh][None, :]
            sc = _dotT(qh, ks[hh]) * scale
            m = jnp.max(sc, axis=-1, keepdims=True)
            p = jnp.exp(sc - m)
            a = p / jnp.sum(p, axis=-1, keepdims=True)
            oh = _dot(a, vs[hh])
            contrib = _dotT(oh, woutr[hh])
            acc = contrib if acc is None else acc + contrib
        blocks.append(acc)
    return jnp.concatenate(blocks, axis=0)


def _bond_layer_body(h_ref, sg_ref, pg_ref, cnt_ref, lng_ref, lnb_ref,
                     w3_ref, b3_ref, woutr_ref, outb_ref, upw_ref, upb_ref,
                     uplng_ref, uplnb_ref, ho_ref):
    h = h_ref[...]
    r = sg_ref[...] - pg_ref[...] - cnt_ref[...] * h
    xn = _ln(r, lng_ref[...], lnb_ref[...])
    attn = _mha_blocks(xn, w3_ref[...], b3_ref[...], woutr_ref[...], _NE)
    t = attn + outb_ref[...] + 2.0 * r
    u = _dotT(t, upw_ref[...]) + upb_ref[...]
    ho_ref[...] = _gelu(_ln(u, uplng_ref[...], uplnb_ref[...]))


def _bond_layer(h, sg, pg, cnt, lng, lnb, in_w, in_b, out_w, out_b,
                up_w, up_b, uplng, uplnb):
    w3 = in_w.reshape(3, _HEADS, _DH, _HID)
    b3 = in_b.reshape(3, _HEADS, _DH)
    woutr = out_w.reshape(_HID, _HEADS, _DH).transpose(1, 0, 2)
    return pl.pallas_call(
        _bond_layer_body,
        out_shape=jax.ShapeDtypeStruct((_NE, _HID), _f32),
    )(h, sg, pg, cnt, lng.reshape(1, -1), lnb.reshape(1, -1),
      w3, b3, woutr, out_b.reshape(1, -1), up_w, up_b.reshape(1, -1),
      uplng.reshape(1, -1), uplnb.reshape(1, -1))


def _atom_body(x_ref, hib_ref, aew_ref, aeb_ref, aelng_ref, aelnb_ref,
               fpwa_ref, fpwb_ref, fpb_ref, alng_ref, alnb_ref,
               a3_ref, ab3_ref, awoutr_ref, aoutb_ref, batch_ref,
               gpw1_ref, gpb1_ref, gplng_ref, gplnb_ref, gpw2_ref, gpb2_ref,
               out_ref):
    a = _gelu(_ln(_dotT(x_ref[...], aew_ref[...]) + aeb_ref[...],
                  aelng_ref[...], aelnb_ref[...]))
    xi = _dotT(a, fpwa_ref[...]) + _dotT(hib_ref[...], fpwb_ref[...]) \
        + fpb_ref[...]
    xn = _ln(xi, alng_ref[...], alnb_ref[...])
    attn = _mha_blocks(xn, a3_ref[...], ab3_ref[...], awoutr_ref[...], _NN)
    hi = attn + aoutb_ref[...] + 2.0 * xi
    gids = lax.broadcasted_iota(jnp.int32, (1, _NG), 1)
    g_oh = (batch_ref[...] == gids).astype(_f32)
    ssum = lax.dot_general(g_oh, hi, (((0,), (0,)), ((), ())),
                           preferred_element_type=_f32)
    ones = jnp.ones((_NN, 1), _f32)
    ccol = lax.dot_general(g_oh, ones, (((0,), (0,)), ((), ())),
                           preferred_element_type=_f32)
    hg = ssum / ccol
    hg2 = _gelu(_ln(_dotT(hg, gpw1_ref[...]) + gpb1_ref[...],
                    gplng_ref[...], gplnb_ref[...]))
    out_ref[...] = _dotT(hg2, gpw2_ref[...]) + gpb2_ref[...]


def _atom_graph(x, hib, batch, ae_w, ae_b, aelng, aelnb, fp_w, fp_b,
                alng, alnb, ain_w, ain_b, aout_w, aout_b,
                gp_w1, gp_b1, gplng, gplnb, gp_w2, gp_b2):
    a3 = ain_w.reshape(3, _HEADS, _DH, _HID)
    ab3 = ain_b.reshape(3, _HEADS, _DH)
    awoutr = aout_w.reshape(_HID, _HEADS, _DH).transpose(1, 0, 2)
    return pl.pallas_call(
        _atom_body,
        out_shape=jax.ShapeDtypeStruct((_NG, _HID), _f32),
    )(x, hib, ae_w, ae_b.reshape(1, -1), aelng.reshape(1, -1),
      aelnb.reshape(1, -1), fp_w[:, :_HID], fp_w[:, _HID:],
      fp_b.reshape(1, -1), alng.reshape(1, -1), alnb.reshape(1, -1),
      a3, ab3, awoutr, aout_b.reshape(1, -1), batch.reshape(_NN, 1),
      gp_w1, gp_b1.reshape(1, -1), gplng.reshape(1, -1),
      gplnb.reshape(1, -1), gp_w2, gp_b2.reshape(1, -1))


def _pair_ids(src, dst):
    key = src * _NN + dst
    order = jnp.argsort(key)
    ks = jnp.take(key, order)
    new = jnp.concatenate([jnp.ones((1,), jnp.int32),
                           (ks[1:] != ks[:-1]).astype(jnp.int32)])
    cids = jnp.cumsum(new) - 1
    return jnp.zeros((_NE,), jnp.int32).at[order].set(cids.astype(jnp.int32))


def kernel(x, edge_attr, edge_index, batch, be_w, be_b, wh_w, wh_b,
           bln_g, bln_b, bin_w, bin_b, bout_w, bout_b, bup_w, bup_b,
           bupln_g, bupln_b, ae_w, ae_b, aeln_g, aeln_b, fp_w, fp_b,
           aln_g, aln_b, ain_w, ain_b, aout_w, aout_b,
           gp_w1, gp_b1, gpln_g, gpln_b, gp_w2, gp_b2):
    src = edge_index[0]
    dst = edge_index[1]
    cid = _pair_ids(src, dst)
    zs = jnp.zeros((_LANES, _NN), _f32)
    zp = jnp.zeros((_LANES, _NE), _f32)

    h, cnt = _embed(edge_attr, be_w, be_b, wh_w, wh_b, dst, src)
    for t in range(_LAYERS):
        sg, pg = _sc_aggregate(h.T, dst, cid, zs, zp)
        h = _bond_layer(h, sg, pg, cnt, bln_g[t], bln_b[t], bin_w[t],
                        bin_b[t], bout_w[t], bout_b[t], bup_w[t], bup_b[t],
                        bupln_g[t], bupln_b[t])
    hib01t = _sc_scatter_nodes(h.T, dst, zs)
    return _atom_graph(x, (hib01t[:_HID] + hib01t[_HID:]).T, batch, ae_w, ae_b,
                       aeln_g, aeln_b, fp_w, fp_b,
                       aln_g, aln_b, ain_w, ain_b, aout_w, aout_b,
                       gp_w1, gp_b1, gpln_g, gpln_b, gp_w2, gp_b2)


# speedup vs baseline: 1.3942x; 1.0857x over previous
import functools

import jax
import jax.numpy as jnp
from jax import lax
from jax.experimental import pallas as pl
from jax.experimental.pallas import tpu as pltpu
from jax.experimental.pallas import tpu_sc as plsc

_NN = 1024
_NE = 2048
_HID = 512
_HEADS = 8
_DH = 64
_LAYERS = 3
_NG = 16
_NC = 2
_NS = 16

_f32 = jnp.float32


def _dotT(a, b):
    return lax.dot_general(a, b, (((1,), (1,)), ((), ())),
                           preferred_element_type=_f32)


def _dot(a, b):
    return lax.dot_general(a, b, (((1,), (0,)), ((), ())),
                           preferred_element_type=_f32)


def _gelu(x):
    return x * 0.5 * (1.0 + lax.erf(x * (2.0 ** -0.5)))


def _ln(x, g, b):
    m = jnp.mean(x, axis=-1, keepdims=True)
    xc = x - m
    v = jnp.mean(xc * xc, axis=-1, keepdims=True)
    return xc * lax.rsqrt(v + 1e-5) * g + b


def _embed_body(ea_ref, bew_ref, beb_ref, whw_ref, whb_ref,
                dstc_ref, dstr_ref, srcc_ref, srcr_ref,
                h_ref, cnt_ref):
    t = _gelu(_dotT(ea_ref[...], bew_ref[...]) + beb_ref[...])
    h_ref[...] = _dotT(t, whw_ref[...]) + whb_ref[...]
    dc = dstc_ref[...]
    dr = dstr_ref[...]
    sc = srcc_ref[...]
    sr = srcr_ref[...]
    blocks = []
    nb = 8
    rb = _NE // nb
    for b in range(nb):
        dcb = dc[b * rb:(b + 1) * rb]
        scb = sc[b * rb:(b + 1) * rb]
        m = ((dcb == dr) & (scb != sr)).astype(_f32)
        blocks.append(jnp.sum(m, axis=-1, keepdims=True))
    cnt_ref[...] = jnp.concatenate(blocks, axis=0)


def _embed(edge_attr, be_w, be_b, wh_w, wh_b, dst, src):
    return pl.pallas_call(
        _embed_body,
        out_shape=(jax.ShapeDtypeStruct((_NE, _HID), _f32),
                   jax.ShapeDtypeStruct((_NE, 1), _f32)),
    )(edge_attr, be_w, be_b.reshape(1, -1), wh_w, wh_b.reshape(1, -1),
      dst.reshape(_NE, 1), dst.reshape(1, _NE),
      src.reshape(_NE, 1), src.reshape(1, _NE))


_LANES = 16
_NW = _NC * _NS
_CH = 64


def _bcast_lane(vec, k):
    idx = jnp.full((_LANES, 1), k, jnp.int32)
    return lax.gather(vec, idx,
                      lax.GatherDimensionNumbers(offset_dims=(),
                                                 collapsed_slice_dims=(0,),
                                                 start_index_map=(0,)),
                      (1,), mode=lax.GatherScatterMode.PROMISE_IN_BOUNDS)


_SC_PARAMS = pltpu.CompilerParams(needs_layout_passes=False)
_G = _NE // _LANES


_SC_PARAMS = pltpu.CompilerParams(needs_layout_passes=False)
_G = _NE // _LANES
_EPC = _NE // _NC
_GC = _EPC // _LANES
_EPW = _NE // (_NC * _NS)


def _bcast_lane(vec, k):
    idx = jnp.full((_LANES, 1), k, jnp.int32)
    return lax.gather(vec, idx,
                      lax.GatherDimensionNumbers(offset_dims=(),
                                                 collapsed_slice_dims=(0,),
                                                 start_index_map=(0,)),
                      (1,), mode=lax.GatherScatterMode.PROMISE_IN_BOUNDS)


def _sc_scat_body(ht_hbm, dst_hbm, cid_hbm, zs_hbm, zp_hbm,
                  s01_hbm, p01_hbm, hbuf, dst_v, cid_v, tab_s, tab_p):
    c = lax.axis_index("c")
    s = lax.axis_index("s")
    col0 = (s * _NC + c) * _LANES
    pltpu.sync_copy(dst_hbm, dst_v)
    pltpu.sync_copy(cid_hbm, cid_v)
    pltpu.sync_copy(ht_hbm.at[pl.ds(col0, _LANES)], hbuf)
    pltpu.sync_copy(zs_hbm, tab_s)
    pltpu.sync_copy(zp_hbm, tab_p)
    lanes = lax.iota(jnp.int32, _LANES)

    def scat(g, cy):
        dvec = dst_v[pl.ds(g * _LANES, _LANES)]
        cvec = cid_v[pl.ds(g * _LANES, _LANES)]
        for k in range(_LANES):
            dcol = _bcast_lane(dvec, k)
            ccol = _bcast_lane(cvec, k)
            ev = jnp.full((_LANES,), g * _LANES + k, jnp.int32)
            hv = plsc.load_gather(hbuf, [lanes, ev])
            plsc.addupdate_scatter(tab_s, [lanes, dcol], hv)
            plsc.addupdate_scatter(tab_p, [lanes, ccol], hv)
        return cy

    lax.fori_loop(c * _GC, (c + 1) * _GC, scat, 0)
    pltpu.sync_copy(tab_s, s01_hbm.at[pl.ds(c * _HID + col0, _LANES)])
    pltpu.sync_copy(tab_p, p01_hbm.at[pl.ds(c * _HID + col0, _LANES)])


def _sc_scatter_partials(ht, dst, cid, zs, zp):
    return pl.kernel(
        _sc_scat_body,
        out_type=(jax.ShapeDtypeStruct((_NC * _HID, _NN), _f32),
                  jax.ShapeDtypeStruct((_NC * _HID, _NE), _f32)),
        mesh=plsc.VectorSubcoreMesh(core_axis_name="c", subcore_axis_name="s"),
        scratch_types=[
            pltpu.VMEM((_LANES, _NE), _f32),
            pltpu.VMEM((_NE,), jnp.int32),
            pltpu.VMEM((_NE,), jnp.int32),
            pltpu.VMEM((_LANES, _NN), _f32),
            pltpu.VMEM((_LANES, _NE), _f32),
        ],
        compiler_params=_SC_PARAMS,
    )(ht, dst, cid, zs, zp)


def _sc_gath_body(dst_hbm, cid_hbm, s_hbm, p_hbm, sg_hbm, pg_hbm,
                  buf, idx1):
    c = lax.axis_index("c")
    s = lax.axis_index("s")
    base = (s * _NC + c) * _EPW
    pltpu.sync_copy(dst_hbm.at[pl.ds(base, _EPW)], idx1)
    pltpu.sync_copy(s_hbm.at[idx1], buf)
    pltpu.sync_copy(buf, sg_hbm.at[pl.ds(base, _EPW)])
    pltpu.sync_copy(cid_hbm.at[pl.ds(base, _EPW)], idx1)
    pltpu.sync_copy(p_hbm.at[idx1], buf)
    pltpu.sync_copy(buf, pg_hbm.at[pl.ds(base, _EPW)])


def _sc_gather_combine(dst, cid, s01, p01):
    return pl.kernel(
        _sc_gath_body,
        out_type=(jax.ShapeDtypeStruct((_NE, _HID), _f32),
                  jax.ShapeDtypeStruct((_NE, _HID), _f32)),
        mesh=plsc.VectorSubcoreMesh(core_axis_name="c", subcore_axis_name="s"),
        scratch_types=[
            pltpu.VMEM((_EPW, _HID), _f32),
            pltpu.VMEM((_EPW,), jnp.int32),
        ],
        compiler_params=_SC_PARAMS,
    )(dst, cid, s01, p01)


def _sc_aggregate(ht, dst, cid, zs, zp):
    s01t, p01t = _sc_scatter_partials(ht, dst, cid, zs, zp)
    s_full = (s01t[:_HID] + s01t[_HID:]).T
    p_full = (p01t[:_HID] + p01t[_HID:]).T
    return _sc_gather_combine(dst, cid, s_full, p_full)


def _sc_nodes_body(ht_hbm, dst_hbm, zs_hbm, hib01_hbm, hbuf, dst_v, tab_s):
    c = lax.axis_index("c")
    s = lax.axis_index("s")
    col0 = (s * _NC + c) * _LANES
    pltpu.sync_copy(dst_hbm, dst_v)
    pltpu.sync_copy(ht_hbm.at[pl.ds(col0, _LANES)], hbuf)
    pltpu.sync_copy(zs_hbm, tab_s)
    lanes = lax.iota(jnp.int32, _LANES)

    def scat(g, cy):
        dvec = dst_v[pl.ds(g * _LANES, _LANES)]
        for k in range(_LANES):
            dcol = _bcast_lane(dvec, k)
            ev = jnp.full((_LANES,), g * _LANES + k, jnp.int32)
            hv = plsc.load_gather(hbuf, [lanes, ev])
            plsc.addupdate_scatter(tab_s, [lanes, dcol], hv)
        return cy

    lax.fori_loop(c * _GC, (c + 1) * _GC, scat, 0)
    pltpu.sync_copy(tab_s, hib01_hbm.at[pl.ds(c * _HID + col0, _LANES)])


def _sc_scatter_nodes(ht, dst, zs):
    return pl.kernel(
        _sc_nodes_body,
        out_type=jax.ShapeDtypeStruct((_NC * _HID, _NN), _f32),
        mesh=plsc.VectorSubcoreMesh(core_axis_name="c", subcore_axis_name="s"),
        scratch_types=[
            pltpu.VMEM((_LANES, _NE), _f32),
            pltpu.VMEM((_NE,), jnp.int32),
            pltpu.VMEM((_LANES, _NN), _f32),
        ],
        compiler_params=_SC_PARAMS,
    )(ht, dst, zs)


_QB = 512


def _mha_blocks(xn, w3, b3, woutr, n):
    ks = []
    vs = []
    for hh in range(_HEADS):
        ks.append(_dotT(xn, w3[1, hh]) + b3[1, hh][None, :])
        vs.append(_dotT(xn, w3[2, hh]) + b3[2, hh][None, :])
    scale = 1.0 / (_DH ** 0.5)
    blocks = []
    for qb in range(n // _QB):
        xb = xn[qb * _QB:(qb + 1) * _QB]
        acc = None
        for hh in range(_HEADS):
            qh = _dotT(xb, w3[0, hh]) + b3[0, hh][None, :]
            sc = _dotT(qh, ks[hh]) * scale
            p = jnp.exp(sc)
            a = p * (1.0 / jnp.sum(p, axis=-1, keepdims=True))
            oh = _dot(a, vs[hh])
            contrib = _dotT(oh, woutr[hh])
            acc = contrib if acc is None else acc + contrib
        blocks.append(acc)
    return jnp.concatenate(blocks, axis=0)


def _bond_layer_body(h_ref, sg_ref, pg_ref, cnt_ref, lng_ref, lnb_ref,
                     w3_ref, b3_ref, woutr_ref, outb_ref, upw_ref, upb_ref,
                     uplng_ref, uplnb_ref, ho_ref):
    h = h_ref[...]
    r = sg_ref[...] - pg_ref[...] - cnt_ref[...] * h
    xn = _ln(r, lng_ref[...], lnb_ref[...])
    attn = _mha_blocks(xn, w3_ref[...], b3_ref[...], woutr_ref[...], _NE)
    t = attn + outb_ref[...] + 2.0 * r
    u = _dotT(t, upw_ref[...]) + upb_ref[...]
    ho_ref[...] = _gelu(_ln(u, uplng_ref[...], uplnb_ref[...]))


def _bond_layer(h, sg, pg, cnt, lng, lnb, in_w, in_b, out_w, out_b,
                up_w, up_b, uplng, uplnb):
    w3 = in_w.reshape(3, _HEADS, _DH, _HID)
    b3 = in_b.reshape(3, _HEADS, _DH)
    woutr = out_w.reshape(_HID, _HEADS, _DH).transpose(1, 0, 2)
    return pl.pallas_call(
        _bond_layer_body,
        out_shape=jax.ShapeDtypeStruct((_NE, _HID), _f32),
    )(h, sg, pg, cnt, lng.reshape(1, -1), lnb.reshape(1, -1),
      w3, b3, woutr, out_b.reshape(1, -1), up_w, up_b.reshape(1, -1),
      uplng.reshape(1, -1), uplnb.reshape(1, -1))


def _atom_body(x_ref, hib_ref, aew_ref, aeb_ref, aelng_ref, aelnb_ref,
               fpwa_ref, fpwb_ref, fpb_ref, alng_ref, alnb_ref,
               a3_ref, ab3_ref, awoutr_ref, aoutb_ref, batch_ref,
               gpw1_ref, gpb1_ref, gplng_ref, gplnb_ref, gpw2_ref, gpb2_ref,
               out_ref):
    a = _gelu(_ln(_dotT(x_ref[...], aew_ref[...]) + aeb_ref[...],
                  aelng_ref[...], aelnb_ref[...]))
    xi = _dotT(a, fpwa_ref[...]) + _dotT(hib_ref[...], fpwb_ref[...]) \
        + fpb_ref[...]
    xn = _ln(xi, alng_ref[...], alnb_ref[...])
    attn = _mha_blocks(xn, a3_ref[...], ab3_ref[...], awoutr_ref[...], _NN)
    hi = attn + aoutb_ref[...] + 2.0 * xi
    gids = lax.broadcasted_iota(jnp.int32, (1, _NG), 1)
    g_oh = (batch_ref[...] == gids).astype(_f32)
    ssum = lax.dot_general(g_oh, hi, (((0,), (0,)), ((), ())),
                           preferred_element_type=_f32)
    ones = jnp.ones((_NN, 1), _f32)
    ccol = lax.dot_general(g_oh, ones, (((0,), (0,)), ((), ())),
                           preferred_element_type=_f32)
    hg = ssum / ccol
    hg2 = _gelu(_ln(_dotT(hg, gpw1_ref[...]) + gpb1_ref[...],
                    gplng_ref[...], gplnb_ref[...]))
    out_ref[...] = _dotT(hg2, gpw2_ref[...]) + gpb2_ref[...]


def _atom_graph(x, hib, batch, ae_w, ae_b, aelng, aelnb, fp_w, fp_b,
                alng, alnb, ain_w, ain_b, aout_w, aout_b,
                gp_w1, gp_b1, gplng, gplnb, gp_w2, gp_b2):
    a3 = ain_w.reshape(3, _HEADS, _DH, _HID)
    ab3 = ain_b.reshape(3, _HEADS, _DH)
    awoutr = aout_w.reshape(_HID, _HEADS, _DH).transpose(1, 0, 2)
    return pl.pallas_call(
        _atom_body,
        out_shape=jax.ShapeDtypeStruct((_NG, _HID), _f32),
    )(x, hib, ae_w, ae_b.reshape(1, -1), aelng.reshape(1, -1),
      aelnb.reshape(1, -1), fp_w[:, :_HID], fp_w[:, _HID:],
      fp_b.reshape(1, -1), alng.reshape(1, -1), alnb.reshape(1, -1),
      a3, ab3, awoutr, aout_b.reshape(1, -1), batch.reshape(_NN, 1),
      gp_w1, gp_b1.reshape(1, -1), gplng.reshape(1, -1),
      gplnb.reshape(1, -1), gp_w2, gp_b2.reshape(1, -1))


def _pair_ids(src, dst):
    key = src * _NN + dst
    order = jnp.argsort(key)
    ks = jnp.take(key, order)
    new = jnp.concatenate([jnp.ones((1,), jnp.int32),
                           (ks[1:] != ks[:-1]).astype(jnp.int32)])
    cids = jnp.cumsum(new) - 1
    return jnp.zeros((_NE,), jnp.int32).at[order].set(cids.astype(jnp.int32))


def kernel(x, edge_attr, edge_index, batch, be_w, be_b, wh_w, wh_b,
           bln_g, bln_b, bin_w, bin_b, bout_w, bout_b, bup_w, bup_b,
           bupln_g, bupln_b, ae_w, ae_b, aeln_g, aeln_b, fp_w, fp_b,
           aln_g, aln_b, ain_w, ain_b, aout_w, aout_b,
           gp_w1, gp_b1, gpln_g, gpln_b, gp_w2, gp_b2):
    src = edge_index[0]
    dst = edge_index[1]
    cid = _pair_ids(src, dst)
    zs = jnp.zeros((_LANES, _NN), _f32)
    zp = jnp.zeros((_LANES, _NE), _f32)

    h, cnt = _embed(edge_attr, be_w, be_b, wh_w, wh_b, dst, src)
    for t in range(_LAYERS):
        sg, pg = _sc_aggregate(h.T, dst, cid, zs, zp)
        h = _bond_layer(h, sg, pg, cnt, bln_g[t], bln_b[t], bin_w[t],
                        bin_b[t], bout_w[t], bout_b[t], bup_w[t], bup_b[t],
                        bupln_g[t], bupln_b[t])
    hib01t = _sc_scatter_nodes(h.T, dst, zs)
    return _atom_graph(x, (hib01t[:_HID] + hib01t[_HID:]).T, batch, ae_w, ae_b,
                       aeln_g, aeln_b, fp_w, fp_b,
                       aln_g, aln_b, ain_w, ain_b, aout_w, aout_b,
                       gp_w1, gp_b1, gpln_g, gpln_b, gp_w2, gp_b2)
